# baseline (speedup 1.0000x reference)
import jax
import jax.numpy as jnp
from jax import lax
from jax.experimental import pallas as pl
from jax.experimental.pallas import tpu as pltpu

_N = 20000
_NB = 20
_NP = _NB * 8 * 128
_K = 300
_IOU_THR = 0.7


def _nms_body(d0, d1, d2, d3, a0, a1, a2, a3, s_in, img, out_ref,
              y1s, x1s, y2s, x2s, live, bmax,
              ky1, kx1, ky2, kx2, karea):
    f32 = jnp.float32
    neg_inf = f32(-jnp.inf)
    h = img[0, 0]
    w = img[0, 1]

    A0, A1, A2, A3 = a0[...], a1[...], a2[...], a3[...]
    heights = A2 - A0
    widths = A3 - A1
    ctr_y = A0 + 0.5 * heights
    ctr_x = A1 + 0.5 * widths
    pred_cy = d0[...] * heights + ctr_y
    pred_cx = d1[...] * widths + ctr_x
    pred_h = jnp.exp(d2[...]) * heights
    pred_w = jnp.exp(d3[...]) * widths
    y1s[...] = jnp.minimum(jnp.maximum(pred_cy - 0.5 * pred_h, 0.0), h)
    x1s[...] = jnp.minimum(jnp.maximum(pred_cx - 0.5 * pred_w, 0.0), w)
    y2s[...] = jnp.minimum(jnp.maximum(pred_cy + 0.5 * pred_h, 0.0), h)
    x2s[...] = jnp.minimum(jnp.maximum(pred_cx + 0.5 * pred_w, 0.0), w)

    s_val = s_in[...]
    live[...] = s_val

    zero_vreg = jnp.zeros((8, 128), f32)
    ky1[...] = zero_vreg
    kx1[...] = zero_vreg
    ky2[...] = zero_vreg
    kx2[...] = zero_vreg
    karea[...] = zero_vreg
    out_ref[...] = jnp.zeros((_K + 4, 128), f32)

    lane1 = lax.broadcasted_iota(jnp.int32, (1, 128), 1)
    sub8 = lax.broadcasted_iota(jnp.int32, (8, 128), 0)
    lane8 = lax.broadcasted_iota(jnp.int32, (8, 128), 1)
    iota_blk = (lax.broadcasted_iota(jnp.int32, (1, 8, 128), 1) * 128
                + lax.broadcasted_iota(jnp.int32, (1, 8, 128), 2))

    bm = jnp.full((1, 128), neg_inf, f32)
    for j in range(_NB):
        bm = jnp.where(lane1 == j, jnp.max(s_val[j]), bm)
    bmax[...] = bm

    def cond(c):
        k, done = c
        return jnp.logical_and(k < _K, jnp.logical_not(done))

    def body(c):
        k, _ = c
        bm = bmax[...]
        m = jnp.max(bm)
        jstar = jnp.min(jnp.where(bm == m, lane1, 128))
        vv = live[pl.ds(jstar, 1)]
        sel = jnp.min(jnp.where(vv == m, iota_blk, 2048))
        onehot = iota_blk == sel
        by1 = jnp.sum(jnp.where(onehot, y1s[pl.ds(jstar, 1)], 0.0))
        bx1 = jnp.sum(jnp.where(onehot, x1s[pl.ds(jstar, 1)], 0.0))
        by2 = jnp.sum(jnp.where(onehot, y2s[pl.ds(jstar, 1)], 0.0))
        bx2 = jnp.sum(jnp.where(onehot, x2s[pl.ds(jstar, 1)], 0.0))
        nv = jnp.where(onehot, neg_inf, vv)
        live[pl.ds(jstar, 1)] = nv
        bmax[...] = jnp.where(lane1 == jstar, jnp.max(nv), bm)

        ka = karea[...]
        yy1 = jnp.maximum(by1, ky1[...])
        xx1 = jnp.maximum(bx1, kx1[...])
        yy2 = jnp.minimum(by2, ky2[...])
        xx2 = jnp.minimum(bx2, kx2[...])
        inter = jnp.maximum(yy2 - yy1, 0.0) * jnp.maximum(xx2 - xx1, 0.0)
        area_b = (by2 - by1) * (bx2 - bx1)
        union = jnp.maximum(area_b + ka - inter, 1e-8)
        iou = inter / union
        sup = jnp.max(jnp.where(iou > _IOU_THR, 1.0, 0.0)) > 0.0
        valid = m > neg_inf
        keep = jnp.logical_and(valid, jnp.logical_not(sup))

        slot = jnp.where(keep, k, 1000)
        msk = jnp.logical_and(sub8 == slot // 128, lane8 == slot % 128)
        z = f32(0.0)
        ky1[...] = jnp.where(msk, jnp.where(keep, by1, z), ky1[...])
        kx1[...] = jnp.where(msk, jnp.where(keep, bx1, z), kx1[...])
        ky2[...] = jnp.where(msk, jnp.where(keep, by2, z), ky2[...])
        kx2[...] = jnp.where(msk, jnp.where(keep, bx2, z), kx2[...])
        karea[...] = jnp.where(msk, jnp.where(keep, area_b, z), ka)

        orow = jnp.where(keep, k, _K + 2)
        rowv = jnp.where(lane1 == 0, by1, z)
        rowv = jnp.where(lane1 == 1, bx1, rowv)
        rowv = jnp.where(lane1 == 2, by2, rowv)
        rowv = jnp.where(lane1 == 3, bx2, rowv)
        out_ref[pl.ds(orow, 1), :] = rowv
        return (k + keep.astype(jnp.int32), jnp.logical_not(valid))

    lax.while_loop(cond, body, (jnp.int32(0), False))


@jax.jit
def kernel(bboxes_txtytwth, anchors, scores, image_shape):
    pad = _NP - _N

    def prep(col):
        return jnp.pad(col, (0, pad)).reshape(_NB, 8, 128)

    args = [prep(bboxes_txtytwth[:, c]) for c in range(4)]
    args += [prep(anchors[:, c]) for c in range(4)]
    args.append(jnp.pad(scores, (0, pad), constant_values=-jnp.inf)
                .reshape(_NB, 8, 128))
    args.append(image_shape.reshape(1, 2))

    vmem3 = pltpu.VMEM((_NB, 8, 128), jnp.float32)
    vreg = pltpu.VMEM((8, 128), jnp.float32)
    out = pl.pallas_call(
        _nms_body,
        out_shape=jax.ShapeDtypeStruct((_K + 4, 128), jnp.float32),
        scratch_shapes=[vmem3] * 5 + [pltpu.VMEM((1, 128), jnp.float32)]
                       + [vreg] * 5,
    )(*args)
    return out[:_K, :4]
